# baseline (speedup 1.0000x reference)
import jax
import jax.numpy as jnp
from jax.experimental import pallas as pl
from jax.experimental.pallas import tpu as pltpu
from jax.experimental.pallas import tpu_sc as plsc

_K = 8192
_D = 256
_BN = 256
_BK = 2048
_GW = 128


_CHUNKS = ((0, 4096), (4096, 4096))


def _rte_bf16(x):
    u = jax.lax.bitcast_convert_type(x, jnp.uint32)
    r = (u + jnp.uint32(0x7FFF) + ((u >> 16) & jnp.uint32(1))) & jnp.uint32(0xFFFF0000)
    return jax.lax.bitcast_convert_type(r, jnp.float32)


def _argmin_body(x_ref, w_ref, a_ref, c_ref, idx_ref):
    x = x_ref[...]
    a = a_ref[...]

    bv = jnp.full((_BN, 1), jnp.inf, jnp.float32)
    bi = jnp.zeros((_BN, 1), jnp.int32)
    lane = jax.lax.broadcasted_iota(jnp.int32, (_BN, 128), 1)
    for start, width in _CHUNKS:
        wk = w_ref[pl.ds(start, width), :]
        ck = c_ref[:, pl.ds(start, width)]
        m2 = jax.lax.dot_general(
            x, wk, dimension_numbers=(((1,), (1,)), ((), ())),
            preferred_element_type=jnp.float32)
        acc = (a - m2[:, 0:128]) + ck[:, 0:128]
        gacc = jnp.zeros((_BN, 128), jnp.int32)
        for g in range(1, width // 128):
            dg = (a - m2[:, g * 128:(g + 1) * 128]) + ck[:, g * 128:(g + 1) * 128]
            pred = dg < acc
            acc = jnp.where(pred, dg, acc)
            gacc = jnp.where(pred, g, gacc)
        cv = jnp.min(acc, axis=1, keepdims=True)
        cand = gacc * 128 + lane + start
        ii = jnp.min(jnp.where(acc == cv, cand, _K), axis=1, keepdims=True)
        take = (cv < bv) | ((cv == bv) & (ii < bi))
        bi = jnp.where(take, ii, bi)
        bv = _rte_bf16(jnp.where(take, cv, bv))
    idx_ref[...] = bi


def _compute_indices(flat, weight, a, c):
    n = flat.shape[0]
    out = pl.pallas_call(
        _argmin_body,
        grid=(n // _BN,),
        in_specs=[
            pl.BlockSpec((_BN, _D), lambda i: (i, 0)),
            pl.BlockSpec((_K, _D), lambda i: (0, 0)),
            pl.BlockSpec((_BN, 1), lambda i: (i, 0)),
            pl.BlockSpec((1, _K), lambda i: (0, 0)),
        ],
        out_specs=pl.BlockSpec((_BN, 1), lambda i: (i, 0)),
        out_shape=jax.ShapeDtypeStruct((n, 1), jnp.int32),
        compiler_params=pltpu.CompilerParams(
            dimension_semantics=("parallel",)),
    )(flat, weight, a, c)
    return out[:, 0]


def _sc_gather(weight, idx_row):
    n = idx_row.shape[1]
    mesh = plsc.VectorSubcoreMesh(core_axis_name="core",
                                  subcore_axis_name="subcore")

    @pl.kernel(out_type=jax.ShapeDtypeStruct((n, _D), weight.dtype),
               mesh=mesh)
    def gather_kernel(w_hbm, i_hbm, o_hbm):
        def body(i_vmem, o_vmem):
            pltpu.sync_copy(w_hbm.at[i_vmem.at[0]], o_vmem)

        pltpu.emit_pipeline(
            body,
            grid=(n // _GW,),
            in_specs=[pl.BlockSpec((1, _GW), lambda i: (0, i))],
            out_specs=[pl.BlockSpec((_GW, _D), lambda i: (i, 0))],
            core_axis_name=("core", "subcore"),
            dimension_semantics=(pltpu.PARALLEL,),
        )(i_hbm, o_hbm)

    return gather_kernel(weight, idx_row)


def kernel(z, weight):
    b, ch, h, w = z.shape
    a = (z ** 2).sum(axis=1).reshape(-1, 1)
    c = (weight.T ** 2).sum(axis=0).reshape(1, _K)
    x_bf = jnp.transpose((2.0 * z).astype(jnp.bfloat16), (0, 2, 3, 1)).reshape(-1, ch)
    idx = _compute_indices(x_bf, weight, a, c)
    qf = _sc_gather(weight, idx.reshape(1, -1))
    q = jnp.transpose(qf.reshape(b, h, w, ch), (0, 3, 1, 2))
    return (q, q, idx.reshape(b, h, w))


# speedup vs baseline: 1.1142x; 1.1142x over previous
import jax
import jax.numpy as jnp
from jax.experimental import pallas as pl
from jax.experimental.pallas import tpu as pltpu
from jax.experimental.pallas import tpu_sc as plsc

_K = 8192
_D = 256
_BN = 512
_BK = 2048
_GW = 128


_CHUNKS = ((0, 4096), (4096, 4096))


def _rte_bf16(x):
    u = jax.lax.bitcast_convert_type(x, jnp.uint32)
    r = (u + jnp.uint32(0x7FFF) + ((u >> 16) & jnp.uint32(1))) & jnp.uint32(0xFFFF0000)
    return jax.lax.bitcast_convert_type(r, jnp.float32)


def _argmin_body(x_ref, w_ref, a_ref, c_ref, idx_ref):
    x = x_ref[...]
    a = a_ref[...]

    bv = jnp.full((_BN, 1), jnp.inf, jnp.float32)
    bi = jnp.zeros((_BN, 1), jnp.int32)
    lane = jax.lax.broadcasted_iota(jnp.int32, (_BN, 128), 1)
    for start, width in _CHUNKS:
        wk = w_ref[pl.ds(start, width), :]
        ck = c_ref[:, pl.ds(start, width)]
        m2 = jax.lax.dot_general(
            x, wk, dimension_numbers=(((1,), (1,)), ((), ())),
            preferred_element_type=jnp.float32)
        acc = (a - m2[:, 0:128]) + ck[:, 0:128]
        gacc = jnp.zeros((_BN, 128), jnp.int32)
        for g in range(1, width // 128):
            dg = (a - m2[:, g * 128:(g + 1) * 128]) + ck[:, g * 128:(g + 1) * 128]
            pred = dg < acc
            acc = jnp.where(pred, dg, acc)
            gacc = jnp.where(pred, g, gacc)
        cv = jnp.min(acc, axis=1, keepdims=True)
        cand = gacc * 128 + lane + start
        ii = jnp.min(jnp.where(acc == cv, cand, _K), axis=1, keepdims=True)
        take = (cv < bv) | ((cv == bv) & (ii < bi))
        bi = jnp.where(take, ii, bi)
        bv = _rte_bf16(jnp.where(take, cv, bv))
    idx_ref[...] = bi


def _compute_indices(flat, weight, a, c):
    n = flat.shape[0]
    out = pl.pallas_call(
        _argmin_body,
        grid=(n // _BN,),
        in_specs=[
            pl.BlockSpec((_BN, _D), lambda i: (i, 0)),
            pl.BlockSpec((_K, _D), lambda i: (0, 0)),
            pl.BlockSpec((_BN, 1), lambda i: (i, 0)),
            pl.BlockSpec((1, _K), lambda i: (0, 0)),
        ],
        out_specs=pl.BlockSpec((_BN, 1), lambda i: (i, 0)),
        out_shape=jax.ShapeDtypeStruct((n, 1), jnp.int32),
        compiler_params=pltpu.CompilerParams(
            dimension_semantics=("parallel",)),
    )(flat, weight, a, c)
    return out[:, 0]


def _sc_gather(weight, idx_row):
    n = idx_row.shape[1]
    mesh = plsc.VectorSubcoreMesh(core_axis_name="core",
                                  subcore_axis_name="subcore")

    @pl.kernel(out_type=jax.ShapeDtypeStruct((n, _D), weight.dtype),
               mesh=mesh)
    def gather_kernel(w_hbm, i_hbm, o_hbm):
        def body(i_vmem, o_vmem):
            pltpu.sync_copy(w_hbm.at[i_vmem.at[0]], o_vmem)

        pltpu.emit_pipeline(
            body,
            grid=(n // _GW,),
            in_specs=[pl.BlockSpec((1, _GW), lambda i: (0, i))],
            out_specs=[pl.BlockSpec((_GW, _D), lambda i: (i, 0))],
            core_axis_name=("core", "subcore"),
            dimension_semantics=(pltpu.PARALLEL,),
        )(i_hbm, o_hbm)

    return gather_kernel(weight, idx_row)


def kernel(z, weight):
    b, ch, h, w = z.shape
    a = (z ** 2).sum(axis=1).reshape(-1, 1)
    c = (weight.T ** 2).sum(axis=0).reshape(1, _K)
    x_bf = jnp.transpose((2.0 * z).astype(jnp.bfloat16), (0, 2, 3, 1)).reshape(-1, ch)
    idx = _compute_indices(x_bf, weight, a, c)
    qf = _sc_gather(weight, idx.reshape(1, -1))
    q = jnp.transpose(qf.reshape(b, h, w, ch), (0, 3, 1, 2))
    return (q, q, idx.reshape(b, h, w))


# speedup vs baseline: 1.1693x; 1.0495x over previous
import jax
import jax.numpy as jnp
from jax.experimental import pallas as pl
from jax.experimental.pallas import tpu as pltpu
from jax.experimental.pallas import tpu_sc as plsc

_K = 8192
_D = 256
_BN = 1024
_BK = 2048
_GW = 128


_CHUNKS = ((0, 4096), (4096, 4096))


def _rte_bf16(x):
    u = jax.lax.bitcast_convert_type(x, jnp.uint32)
    r = (u + jnp.uint32(0x7FFF) + ((u >> 16) & jnp.uint32(1))) & jnp.uint32(0xFFFF0000)
    return jax.lax.bitcast_convert_type(r, jnp.float32)


def _argmin_body(x_ref, w_ref, a_ref, c_ref, idx_ref):
    x = x_ref[...]
    a = a_ref[...]

    bv = jnp.full((_BN, 1), jnp.inf, jnp.float32)
    bi = jnp.zeros((_BN, 1), jnp.int32)
    lane = jax.lax.broadcasted_iota(jnp.int32, (_BN, 128), 1)
    for start, width in _CHUNKS:
        wk = w_ref[pl.ds(start, width), :]
        ck = c_ref[:, pl.ds(start, width)]
        m2 = jax.lax.dot_general(
            x, wk, dimension_numbers=(((1,), (1,)), ((), ())),
            preferred_element_type=jnp.float32)
        acc = (a - m2[:, 0:128]) + ck[:, 0:128]
        gacc = jnp.zeros((_BN, 128), jnp.int32)
        for g in range(1, width // 128):
            dg = (a - m2[:, g * 128:(g + 1) * 128]) + ck[:, g * 128:(g + 1) * 128]
            pred = dg < acc
            acc = jnp.where(pred, dg, acc)
            gacc = jnp.where(pred, g, gacc)
        cv = jnp.min(acc, axis=1, keepdims=True)
        cand = gacc * 128 + lane + start
        ii = jnp.min(jnp.where(acc == cv, cand, _K), axis=1, keepdims=True)
        take = (cv < bv) | ((cv == bv) & (ii < bi))
        bi = jnp.where(take, ii, bi)
        bv = _rte_bf16(jnp.where(take, cv, bv))
    idx_ref[...] = bi


def _compute_indices(flat, weight, a, c):
    n = flat.shape[0]
    out = pl.pallas_call(
        _argmin_body,
        grid=(n // _BN,),
        in_specs=[
            pl.BlockSpec((_BN, _D), lambda i: (i, 0)),
            pl.BlockSpec((_K, _D), lambda i: (0, 0)),
            pl.BlockSpec((_BN, 1), lambda i: (i, 0)),
            pl.BlockSpec((1, _K), lambda i: (0, 0)),
        ],
        out_specs=pl.BlockSpec((_BN, 1), lambda i: (i, 0)),
        out_shape=jax.ShapeDtypeStruct((n, 1), jnp.int32),
        compiler_params=pltpu.CompilerParams(
            dimension_semantics=("parallel",)),
    )(flat, weight, a, c)
    return out[:, 0]


def _sc_gather(weight, idx_row):
    n = idx_row.shape[1]
    mesh = plsc.VectorSubcoreMesh(core_axis_name="core",
                                  subcore_axis_name="subcore")

    @pl.kernel(out_type=jax.ShapeDtypeStruct((n, _D), weight.dtype),
               mesh=mesh)
    def gather_kernel(w_hbm, i_hbm, o_hbm):
        def body(i_vmem, o_vmem):
            pltpu.sync_copy(w_hbm.at[i_vmem.at[0]], o_vmem)

        pltpu.emit_pipeline(
            body,
            grid=(n // _GW,),
            in_specs=[pl.BlockSpec((1, _GW), lambda i: (0, i))],
            out_specs=[pl.BlockSpec((_GW, _D), lambda i: (i, 0))],
            core_axis_name=("core", "subcore"),
            dimension_semantics=(pltpu.PARALLEL,),
        )(i_hbm, o_hbm)

    return gather_kernel(weight, idx_row)


def kernel(z, weight):
    b, ch, h, w = z.shape
    a = (z ** 2).sum(axis=1).reshape(-1, 1)
    c = (weight.T ** 2).sum(axis=0).reshape(1, _K)
    x_bf = jnp.transpose((2.0 * z).astype(jnp.bfloat16), (0, 2, 3, 1)).reshape(-1, ch)
    idx = _compute_indices(x_bf, weight, a, c)
    qf = _sc_gather(weight, idx.reshape(1, -1))
    q = jnp.transpose(qf.reshape(b, h, w, ch), (0, 3, 1, 2))
    return (q, q, idx.reshape(b, h, w))


# speedup vs baseline: 1.2247x; 1.0474x over previous
import jax
import jax.numpy as jnp
from jax.experimental import pallas as pl
from jax.experimental.pallas import tpu as pltpu
from jax.experimental.pallas import tpu_sc as plsc

_K = 8192
_D = 256
_BN = 2048
_BK = 2048
_GW = 128


_CHUNKS = ((0, 4096), (4096, 4096))


def _rte_bf16(x):
    u = jax.lax.bitcast_convert_type(x, jnp.uint32)
    r = (u + jnp.uint32(0x7FFF) + ((u >> 16) & jnp.uint32(1))) & jnp.uint32(0xFFFF0000)
    return jax.lax.bitcast_convert_type(r, jnp.float32)


def _argmin_body(x_ref, w_ref, a_ref, c_ref, idx_ref):
    x = x_ref[...]
    a = a_ref[...]

    bv = jnp.full((_BN, 1), jnp.inf, jnp.float32)
    bi = jnp.zeros((_BN, 1), jnp.int32)
    lane = jax.lax.broadcasted_iota(jnp.int32, (_BN, 128), 1)
    for start, width in _CHUNKS:
        wk = w_ref[pl.ds(start, width), :]
        ck = c_ref[:, pl.ds(start, width)]
        m2 = jax.lax.dot_general(
            x, wk, dimension_numbers=(((1,), (1,)), ((), ())),
            preferred_element_type=jnp.float32)
        acc = (a - m2[:, 0:128]) + ck[:, 0:128]
        gacc = jnp.zeros((_BN, 128), jnp.int32)
        for g in range(1, width // 128):
            dg = (a - m2[:, g * 128:(g + 1) * 128]) + ck[:, g * 128:(g + 1) * 128]
            pred = dg < acc
            acc = jnp.where(pred, dg, acc)
            gacc = jnp.where(pred, g, gacc)
        cv = jnp.min(acc, axis=1, keepdims=True)
        cand = gacc * 128 + lane + start
        ii = jnp.min(jnp.where(acc == cv, cand, _K), axis=1, keepdims=True)
        take = (cv < bv) | ((cv == bv) & (ii < bi))
        bi = jnp.where(take, ii, bi)
        bv = _rte_bf16(jnp.where(take, cv, bv))
    idx_ref[...] = bi


def _compute_indices(flat, weight, a, c):
    n = flat.shape[0]
    out = pl.pallas_call(
        _argmin_body,
        grid=(n // _BN,),
        in_specs=[
            pl.BlockSpec((_BN, _D), lambda i: (i, 0)),
            pl.BlockSpec((_K, _D), lambda i: (0, 0)),
            pl.BlockSpec((_BN, 1), lambda i: (i, 0)),
            pl.BlockSpec((1, _K), lambda i: (0, 0)),
        ],
        out_specs=pl.BlockSpec((_BN, 1), lambda i: (i, 0)),
        out_shape=jax.ShapeDtypeStruct((n, 1), jnp.int32),
        compiler_params=pltpu.CompilerParams(
            dimension_semantics=("parallel",)),
    )(flat, weight, a, c)
    return out[:, 0]


def _sc_gather(weight, idx_row):
    n = idx_row.shape[1]
    mesh = plsc.VectorSubcoreMesh(core_axis_name="core",
                                  subcore_axis_name="subcore")

    @pl.kernel(out_type=jax.ShapeDtypeStruct((n, _D), weight.dtype),
               mesh=mesh)
    def gather_kernel(w_hbm, i_hbm, o_hbm):
        def body(i_vmem, o_vmem):
            pltpu.sync_copy(w_hbm.at[i_vmem.at[0]], o_vmem)

        pltpu.emit_pipeline(
            body,
            grid=(n // _GW,),
            in_specs=[pl.BlockSpec((1, _GW), lambda i: (0, i))],
            out_specs=[pl.BlockSpec((_GW, _D), lambda i: (i, 0))],
            core_axis_name=("core", "subcore"),
            dimension_semantics=(pltpu.PARALLEL,),
        )(i_hbm, o_hbm)

    return gather_kernel(weight, idx_row)


def kernel(z, weight):
    b, ch, h, w = z.shape
    a = (z ** 2).sum(axis=1).reshape(-1, 1)
    c = (weight.T ** 2).sum(axis=0).reshape(1, _K)
    x_bf = jnp.transpose((2.0 * z).astype(jnp.bfloat16), (0, 2, 3, 1)).reshape(-1, ch)
    idx = _compute_indices(x_bf, weight, a, c)
    qf = _sc_gather(weight, idx.reshape(1, -1))
    q = jnp.transpose(qf.reshape(b, h, w, ch), (0, 3, 1, 2))
    return (q, q, idx.reshape(b, h, w))
